# Initial kernel scaffold; baseline (speedup 1.0000x reference)
import jax
import jax.numpy as jnp
from jax.experimental import pallas as pl


def kernel(x, edge_index, edge_attr, batch, W_edge, W_msg, b_msg, W_pred, b_pred):
    raise NotImplementedError("write your pallas kernel here")


# speedup vs baseline: 2.6741x; 2.6741x over previous
import functools

import jax
import jax.numpy as jnp
from jax import lax
from jax.experimental import pallas as pl
from jax.experimental.pallas import tpu as pltpu
from jax.experimental.pallas import tpu_sc as plsc

_NC = 2
_NS = 16
_NW = _NC * _NS
_CHUNK = 128
_G = 128


def _sc_aggregate(x_p, src_p, dstr, ea_p, n):
    d = x_p.shape[1]
    de = ea_p.shape[1]
    e_pad = src_p.shape[0]
    per_t = e_pad // _NS
    n_chunks = per_t // _CHUNK
    n_acc = -(-n // 8) * 8
    rpt = -(-n_acc // (8 * _NS)) * 8
    n_zd = -(-rpt // _CHUNK)

    mesh = plsc.VectorSubcoreMesh(core_axis_name="c", subcore_axis_name="s")

    @functools.partial(
        pl.kernel,
        out_type=[
            jax.ShapeDtypeStruct((n_acc, d), jnp.float32),
            jax.ShapeDtypeStruct((n_acc, d), jnp.float32),
        ],
        mesh=mesh,
        scratch_types=[
            pltpu.VMEM((_CHUNK,), jnp.int32),
            pltpu.VMEM((2, 64), jnp.int32),
            pltpu.VMEM((_CHUNK, d), jnp.float32),
            pltpu.VMEM((64, de), jnp.float32),
            pltpu.VMEM((64, d), jnp.float32),
            pltpu.VMEM_SHARED((n_acc, d), jnp.float32),
            pltpu.SemaphoreType.DMA,
        ],
    )
    def k(x_hbm, src_hbm, dstr_hbm, ea_hbm, outd_hbm, oute_hbm,
          srcv, dst2, gbuf, ebuf32, ebuf, acc, sem):
        c = lax.axis_index("c")
        s = lax.axis_index("s")
        z16 = jnp.zeros((16,), jnp.float32)
        iota = lax.iota(jnp.int32, 16)

        def ident128(base_r):
            for j in range(_CHUNK // 16):
                srcv[pl.ds(j * 16, 16)] = base_r + j * 16 + iota

        @pl.loop(0, _CHUNK)
        def _(r):
            for j in range(d // 16):
                gbuf[r, pl.ds(j * 16, 16)] = z16

        @pl.loop(0, 64)
        def _(r):
            for j in range(d // 16):
                ebuf[r, pl.ds(j * 16, 16)] = z16

        @pl.loop(0, n_zd)
        def _(k_):
            ident128(jnp.minimum(s * rpt + k_ * _CHUNK, n_acc - _CHUNK))
            pltpu.sync_copy(gbuf, acc.at[srcv])

        plsc.subcore_barrier()

        @pl.when(c == 0)
        def _():
            @pl.loop(0, n_chunks)
            def _(i):
                base = s * per_t + i * _CHUNK
                pltpu.sync_copy(src_hbm.at[pl.ds(base, _CHUNK)], srcv)
                pltpu.sync_copy(
                    dstr_hbm.at[pl.ds(s * (per_t // 64) + i * 2, 2)], dst2)
                pltpu.async_copy(x_hbm.at[srcv], gbuf, sem).wait()
                for j in range(2):
                    pltpu.sync_copy(gbuf.at[pl.ds(j * 64, 64)],
                                    acc.at[dst2.at[j]], add=True)

        @pl.when(c == 1)
        def _():
            @pl.loop(0, n_chunks)
            def _(i):
                base = s * per_t + i * _CHUNK
                pltpu.sync_copy(
                    dstr_hbm.at[pl.ds(s * (per_t // 64) + i * 2, 2)], dst2)
                for j in range(2):
                    pltpu.sync_copy(ea_hbm.at[pl.ds(base + j * 64, 64)],
                                    ebuf32)

                    @pl.loop(0, 64)
                    def _(r):
                        for q in range(de // 16):
                            ebuf[r, pl.ds(q * 16, 16)] = (
                                ebuf32[r, pl.ds(q * 16, 16)])

                    pltpu.sync_copy(ebuf, acc.at[dst2.at[j]], add=True)

        plsc.subcore_barrier()

        @pl.loop(0, n_zd)
        def _(k_):
            base_r = jnp.minimum(s * rpt + k_ * _CHUNK, n_acc - _CHUNK)
            ident128(base_r)
            pltpu.async_copy(acc.at[srcv], gbuf, sem).wait()

            @pl.when(c == 0)
            def _():
                pltpu.sync_copy(gbuf, outd_hbm.at[pl.ds(base_r, _CHUNK)])

            @pl.when(c == 1)
            def _():
                pltpu.sync_copy(gbuf, oute_hbm.at[pl.ds(base_r, _CHUNK)])

    return k(x_p, src_p, dstr, ea_p)


def _tc_head(x, aggd, agge, batch3, W_edge, W_msg, b_msg2, W_pred, b_pred2):
    n, d = x.shape
    de = W_edge.shape[0]
    t = W_pred.shape[1]
    nb = 1000
    grid = n // nb
    f32 = jnp.float32
    hi = lax.Precision.HIGHEST

    def body(x_ref, ad_ref, ae_ref, b_ref, we_ref, wm_ref, bm_ref, wp_ref,
             bp_ref, out_ref, acc, cnt):
        i = pl.program_id(0)

        @pl.when(i == 0)
        def _():
            acc[...] = jnp.zeros_like(acc)
            cnt[...] = jnp.zeros_like(cnt)

        a = x_ref[...] + ad_ref[...]
        h = a + jnp.dot(ae_ref[...], we_ref[...], preferred_element_type=f32,
                        precision=hi)
        z = jnp.dot(h, wm_ref[...], preferred_element_type=f32,
                    precision=hi) + bm_ref[...]
        r = jnp.maximum(z, 0.0)
        b2 = b_ref[0]
        p = (lax.broadcasted_iota(jnp.int32, (_G, nb), 0) == b2).astype(f32)
        acc[...] += jnp.dot(p, r, preferred_element_type=f32, precision=hi)
        cnt[...] += jnp.broadcast_to(jnp.sum(p, axis=1, keepdims=True),
                                     cnt.shape)

        @pl.when(i == grid - 1)
        def _():
            g = acc[...] / jnp.clip(cnt[...], 1.0, None)
            out_ref[...] = jnp.dot(g, wp_ref[...], preferred_element_type=f32,
                                   precision=hi) + bp_ref[...]

    return pl.pallas_call(
        body,
        grid=(grid,),
        in_specs=[
            pl.BlockSpec((nb, d), lambda i: (i, 0)),
            pl.BlockSpec((nb, d), lambda i: (i, 0)),
            pl.BlockSpec((nb, de), lambda i: (i, 0)),
            pl.BlockSpec((1, 1, nb), lambda i: (i, 0, 0)),
            pl.BlockSpec((de, d), lambda i: (0, 0)),
            pl.BlockSpec((d, d), lambda i: (0, 0)),
            pl.BlockSpec((1, d), lambda i: (0, 0)),
            pl.BlockSpec((d, t), lambda i: (0, 0)),
            pl.BlockSpec((1, t), lambda i: (0, 0)),
        ],
        out_specs=pl.BlockSpec((_G, t), lambda i: (0, 0)),
        out_shape=jax.ShapeDtypeStruct((_G, t), f32),
        scratch_shapes=[
            pltpu.VMEM((_G, d), f32),
            pltpu.VMEM((_G, d), f32),
        ],
    )(x, aggd, agge, batch3, W_edge, W_msg, b_msg2, W_pred, b_pred2)


def kernel(x, edge_index, edge_attr, batch, W_edge, W_msg, b_msg, W_pred,
           b_pred):
    n, d = x.shape
    e = edge_index.shape[1]
    de = edge_attr.shape[1]
    t = W_pred.shape[1]

    stride = _NS * _CHUNK
    e_pad = ((e + stride - 1) // stride) * stride
    pad = e_pad - e

    x_p = jnp.pad(x, ((0, 8), (0, 0)))
    src_p = jnp.concatenate([edge_index[0],
                             jnp.full((pad,), n, jnp.int32)])
    dst_p = jnp.concatenate([edge_index[1],
                             jnp.zeros((pad,), jnp.int32)])
    dstr = dst_p.reshape(-1, 64)
    ea_p = jnp.pad(edge_attr, ((0, pad), (0, 0)))
    We_p = jnp.pad(W_edge, ((0, d - de), (0, 0)))
    aggd, agge = _sc_aggregate(x_p, src_p, dstr, ea_p, n)

    batch3 = batch.reshape(n // 1000, 1, 1000)
    out = _tc_head(x, aggd, agge, batch3, We_p, W_msg,
                   b_msg.reshape(1, d), W_pred, b_pred.reshape(1, t))
    return out


# speedup vs baseline: 2.8680x; 1.0725x over previous
import functools

import jax
import jax.numpy as jnp
from jax import lax
from jax.experimental import pallas as pl
from jax.experimental.pallas import tpu as pltpu
from jax.experimental.pallas import tpu_sc as plsc

_NC = 2
_NS = 16
_NW = _NC * _NS
_CHUNK = 128
_G = 128


def _sc_aggregate(x_p, src_p, dstr, ea_p, n):
    d = x_p.shape[1]
    de = ea_p.shape[1]
    e_pad = src_p.shape[0]
    per_t = e_pad // _NS
    n_chunks = per_t // _CHUNK
    n_acc = -(-n // 8) * 8
    rpt = -(-n_acc // (8 * _NS)) * 8
    n_zd = -(-rpt // _CHUNK)

    mesh = plsc.VectorSubcoreMesh(core_axis_name="c", subcore_axis_name="s")

    @functools.partial(
        pl.kernel,
        out_type=[
            jax.ShapeDtypeStruct((n_acc, d), jnp.float32),
            jax.ShapeDtypeStruct((n_acc, d), jnp.float32),
        ],
        mesh=mesh,
        scratch_types=[
            pltpu.VMEM((_CHUNK,), jnp.int32),
            pltpu.VMEM((2, 64), jnp.int32),
            pltpu.VMEM((_CHUNK, d), jnp.float32),
            pltpu.VMEM((64, de), jnp.float32),
            pltpu.VMEM((64, d), jnp.float32),
            pltpu.VMEM_SHARED((n_acc, d), jnp.float32),
            pltpu.SemaphoreType.DMA,
            pltpu.SemaphoreType.DMA,
        ],
    )
    def k(x_hbm, src_hbm, dstr_hbm, ea_hbm, outd_hbm, oute_hbm,
          srcv, dst2, gbuf, ebuf32, ebuf, acc, sem, sem2):
        c = lax.axis_index("c")
        s = lax.axis_index("s")
        z16 = jnp.zeros((16,), jnp.float32)
        iota = lax.iota(jnp.int32, 16)

        def ident128(base_r):
            for j in range(_CHUNK // 16):
                srcv[pl.ds(j * 16, 16)] = base_r + j * 16 + iota

        @pl.loop(0, _CHUNK)
        def _(r):
            for j in range(d // 16):
                gbuf[r, pl.ds(j * 16, 16)] = z16

        @pl.loop(0, 64)
        def _(r):
            for j in range(d // 16):
                ebuf[r, pl.ds(j * 16, 16)] = z16

        @pl.loop(0, n_zd)
        def _(k_):
            ident128(jnp.minimum(s * rpt + k_ * _CHUNK, n_acc - _CHUNK))
            pltpu.sync_copy(gbuf, acc.at[srcv])

        plsc.subcore_barrier()

        @pl.when(c == 0)
        def _():
            @pl.loop(0, n_chunks)
            def _(i):
                base = s * per_t + i * _CHUNK
                pltpu.sync_copy(src_hbm.at[pl.ds(base, _CHUNK)], srcv)
                hg = pltpu.async_copy(x_hbm.at[srcv], gbuf, sem)
                pltpu.sync_copy(
                    dstr_hbm.at[pl.ds(s * (per_t // 64) + i * 2, 2)], dst2)
                hg.wait()
                hs = [pltpu.async_copy(gbuf.at[pl.ds(j * 64, 64)],
                                       acc.at[dst2.at[j]], sem2, add=True)
                      for j in range(2)]
                for h in hs:
                    h.wait()

        @pl.when(c == 1)
        def _():
            @pl.loop(0, n_chunks)
            def _(i):
                base = s * per_t + i * _CHUNK
                pltpu.sync_copy(
                    dstr_hbm.at[pl.ds(s * (per_t // 64) + i * 2, 2)], dst2)
                pltpu.sync_copy(ea_hbm.at[pl.ds(base, 64)], ebuf32)

                @pl.loop(0, 64)
                def _(r):
                    for q in range(de // 16):
                        ebuf[r, pl.ds(q * 16, 16)] = (
                            ebuf32[r, pl.ds(q * 16, 16)])

                hs = pltpu.async_copy(ebuf, acc.at[dst2.at[0]], sem2,
                                      add=True)
                pltpu.sync_copy(ea_hbm.at[pl.ds(base + 64, 64)], ebuf32)
                hs.wait()

                @pl.loop(0, 64)
                def _(r):
                    for q in range(de // 16):
                        ebuf[r, pl.ds(q * 16, 16)] = (
                            ebuf32[r, pl.ds(q * 16, 16)])

                pltpu.sync_copy(ebuf, acc.at[dst2.at[1]], add=True)

        plsc.subcore_barrier()

        @pl.loop(0, n_zd)
        def _(k_):
            base_r = jnp.minimum(s * rpt + k_ * _CHUNK, n_acc - _CHUNK)
            ident128(base_r)
            pltpu.async_copy(acc.at[srcv], gbuf, sem).wait()

            @pl.when(c == 0)
            def _():
                pltpu.sync_copy(gbuf, outd_hbm.at[pl.ds(base_r, _CHUNK)])

            @pl.when(c == 1)
            def _():
                pltpu.sync_copy(gbuf, oute_hbm.at[pl.ds(base_r, _CHUNK)])

    return k(x_p, src_p, dstr, ea_p)


def _tc_head(x, aggd, agge, batch3, W_edge, W_msg, b_msg2, W_pred, b_pred2):
    n, d = x.shape
    de = W_edge.shape[0]
    t = W_pred.shape[1]
    nb = 1000
    grid = n // nb
    f32 = jnp.float32
    hi = lax.Precision.HIGHEST

    def body(x_ref, ad_ref, ae_ref, b_ref, we_ref, wm_ref, bm_ref, wp_ref,
             bp_ref, out_ref, acc, cnt):
        i = pl.program_id(0)

        @pl.when(i == 0)
        def _():
            acc[...] = jnp.zeros_like(acc)
            cnt[...] = jnp.zeros_like(cnt)

        a = x_ref[...] + ad_ref[...]
        h = a + jnp.dot(ae_ref[...], we_ref[...], preferred_element_type=f32,
                        precision=hi)
        z = jnp.dot(h, wm_ref[...], preferred_element_type=f32,
                    precision=hi) + bm_ref[...]
        r = jnp.maximum(z, 0.0)
        b2 = b_ref[0]
        p = (lax.broadcasted_iota(jnp.int32, (_G, nb), 0) == b2).astype(f32)
        acc[...] += jnp.dot(p, r, preferred_element_type=f32, precision=hi)
        cnt[...] += jnp.broadcast_to(jnp.sum(p, axis=1, keepdims=True),
                                     cnt.shape)

        @pl.when(i == grid - 1)
        def _():
            g = acc[...] / jnp.clip(cnt[...], 1.0, None)
            out_ref[...] = jnp.dot(g, wp_ref[...], preferred_element_type=f32,
                                   precision=hi) + bp_ref[...]

    return pl.pallas_call(
        body,
        grid=(grid,),
        in_specs=[
            pl.BlockSpec((nb, d), lambda i: (i, 0)),
            pl.BlockSpec((nb, d), lambda i: (i, 0)),
            pl.BlockSpec((nb, de), lambda i: (i, 0)),
            pl.BlockSpec((1, 1, nb), lambda i: (i, 0, 0)),
            pl.BlockSpec((de, d), lambda i: (0, 0)),
            pl.BlockSpec((d, d), lambda i: (0, 0)),
            pl.BlockSpec((1, d), lambda i: (0, 0)),
            pl.BlockSpec((d, t), lambda i: (0, 0)),
            pl.BlockSpec((1, t), lambda i: (0, 0)),
        ],
        out_specs=pl.BlockSpec((_G, t), lambda i: (0, 0)),
        out_shape=jax.ShapeDtypeStruct((_G, t), f32),
        scratch_shapes=[
            pltpu.VMEM((_G, d), f32),
            pltpu.VMEM((_G, d), f32),
        ],
    )(x, aggd, agge, batch3, W_edge, W_msg, b_msg2, W_pred, b_pred2)


def kernel(x, edge_index, edge_attr, batch, W_edge, W_msg, b_msg, W_pred,
           b_pred):
    n, d = x.shape
    e = edge_index.shape[1]
    de = edge_attr.shape[1]
    t = W_pred.shape[1]

    stride = _NS * _CHUNK
    e_pad = ((e + stride - 1) // stride) * stride
    pad = e_pad - e

    x_p = jnp.pad(x, ((0, 8), (0, 0)))
    src_p = jnp.concatenate([edge_index[0],
                             jnp.full((pad,), n, jnp.int32)])
    dst_p = jnp.concatenate([edge_index[1],
                             jnp.zeros((pad,), jnp.int32)])
    dstr = dst_p.reshape(-1, 64)
    ea_p = jnp.pad(edge_attr, ((0, pad), (0, 0)))
    We_p = jnp.pad(W_edge, ((0, d - de), (0, 0)))
    aggd, agge = _sc_aggregate(x_p, src_p, dstr, ea_p, n)

    batch3 = batch.reshape(n // 1000, 1, 1000)
    out = _tc_head(x, aggd, agge, batch3, We_p, W_msg,
                   b_msg.reshape(1, d), W_pred, b_pred.reshape(1, t))
    return out
